# baseline (speedup 1.0000x reference)
import functools

import jax
import jax.numpy as jnp
from jax import lax
from jax.experimental import pallas as pl
from jax.experimental.pallas import tpu as pltpu
from jax.experimental.pallas import tpu_sc as plsc

_INFO = plsc.get_sparse_core_info()
_NC = _INFO.num_cores
_NS = _INFO.num_subcores
_NW = _NC * _NS

_CHUNK = 16
_PHASES = 2


def _make_sc_kernel(n_rows, hidden):
    rows_per_w = n_rows // _NW
    n_chunks = rows_per_w // _CHUNK
    vecs_per_row = hidden // 16
    mesh = plsc.VectorSubcoreMesh(core_axis_name="c", subcore_axis_name="s")

    @functools.partial(
        pl.kernel,
        mesh=mesh,
        out_type=jax.ShapeDtypeStruct((n_rows, hidden), jnp.float32),
        scratch_types=[
            pltpu.VMEM((rows_per_w,), jnp.int32),
            pltpu.VMEM((rows_per_w,), jnp.int32),
            pltpu.VMEM((2, _CHUNK, hidden), jnp.float32),
            pltpu.VMEM((2, _CHUNK, hidden), jnp.float32),
            pltpu.VMEM((2, _CHUNK, hidden), jnp.float32),
            pltpu.SemaphoreType.DMA,
            pltpu.SemaphoreType.DMA,
            pltpu.SemaphoreType.DMA,
            pltpu.SemaphoreType.DMA,
            pltpu.SemaphoreType.DMA,
            pltpu.SemaphoreType.DMA,
        ],
    )
    def k(widx_hbm, pidx_hbm, word_hbm, pos_hbm, out_hbm,
          widx_v, pidx_v, wbuf, pbuf, obuf,
          sem_w0, sem_w1, sem_p0, sem_p1, sem_o0, sem_o1):
        sem_w = (sem_w0, sem_w1)
        sem_p = (sem_p0, sem_p1)
        sem_o = (sem_o0, sem_o1)
        wid = lax.axis_index("s") * _NC + lax.axis_index("c")
        base = wid * rows_per_w

        pltpu.sync_copy(widx_hbm.at[pl.ds(base, rows_per_w)], widx_v)
        pltpu.sync_copy(pidx_hbm.at[pl.ds(base, rows_per_w)], pidx_v)

        def start_gathers(c, b):
            idx = pl.ds(c * _CHUNK, _CHUNK)
            pltpu.async_copy(word_hbm.at[widx_v.at[idx]], wbuf.at[b], sem_w[b])
            pltpu.async_copy(pos_hbm.at[pidx_v.at[idx]], pbuf.at[b], sem_p[b])

        def wait_gathers(c, b):
            idx = pl.ds(c * _CHUNK, _CHUNK)
            pltpu.make_async_copy(word_hbm.at[widx_v.at[idx]], wbuf.at[b], sem_w[b]).wait()
            pltpu.make_async_copy(pos_hbm.at[pidx_v.at[idx]], pbuf.at[b], sem_p[b]).wait()

        def out_slice(c):
            return out_hbm.at[pl.ds(base + c * _CHUNK, _CHUNK)]

        for b in range(2):
            start_gathers(b, b)

        def chunk_pair(g, _):
            for b in range(2):
                c = g * 2 + b
                wait_gathers(c, b)

                @pl.when(c >= 2)
                def _():
                    pltpu.make_async_copy(obuf.at[b], out_slice(c - 2), sem_o[b]).wait()

                def add_body(i, _):
                    r = i // vecs_per_row
                    j = (i % vecs_per_row) * 16
                    obuf[b, r, pl.ds(j, 16)] = (
                        wbuf[b, r, pl.ds(j, 16)] + pbuf[b, r, pl.ds(j, 16)]
                    )
                    return 0

                lax.fori_loop(0, _CHUNK * vecs_per_row, add_body, 0, unroll=8)

                pltpu.async_copy(obuf.at[b], out_slice(c), sem_o[b])

                @pl.when(c + 2 < n_chunks)
                def _():
                    start_gathers(c + 2, b)
            return 0

        lax.fori_loop(0, n_chunks // 2, chunk_pair, 0)

        for b in range(2):
            c = n_chunks - 2 + b
            pltpu.make_async_copy(obuf.at[b], out_slice(c), sem_o[b]).wait()

    return k


def kernel(input_ids, position_ids, word_embeddings, position_embeddings):
    batch, seq = input_ids.shape
    hidden = word_embeddings.shape[1]
    n_rows = batch * seq

    widx = jnp.transpose(input_ids, (1, 0)).reshape(n_rows).astype(jnp.int32)
    pidx = jnp.transpose(position_ids, (1, 0)).reshape(n_rows).astype(jnp.int32)

    rows_per_phase = n_rows // _PHASES
    s_per_phase = seq // _PHASES
    k = _make_sc_kernel(rows_per_phase, hidden)

    parts = []
    for p in range(_PHASES):
        sl = slice(p * rows_per_phase, (p + 1) * rows_per_phase)
        flat = k(widx[sl], pidx[sl], word_embeddings, position_embeddings)
        parts.append(flat.reshape(s_per_phase, batch, hidden))
    return jnp.concatenate(parts, axis=0)


# speedup vs baseline: 1.0762x; 1.0762x over previous
import functools

import jax
import jax.numpy as jnp
from jax import lax
from jax.experimental import pallas as pl
from jax.experimental.pallas import tpu as pltpu
from jax.experimental.pallas import tpu_sc as plsc

_INFO = plsc.get_sparse_core_info()
_NC = _INFO.num_cores
_NS = _INFO.num_subcores
_NW = _NC * _NS

_CHUNK = 16
_PHASES = 4


def _make_sc_kernel(n_rows, hidden):
    rows_per_w = n_rows // _NW
    n_chunks = rows_per_w // _CHUNK
    vecs_per_row = hidden // 16
    mesh = plsc.VectorSubcoreMesh(core_axis_name="c", subcore_axis_name="s")

    @functools.partial(
        pl.kernel,
        mesh=mesh,
        out_type=jax.ShapeDtypeStruct((n_rows, hidden), jnp.float32),
        scratch_types=[
            pltpu.VMEM((rows_per_w,), jnp.int32),
            pltpu.VMEM((rows_per_w,), jnp.int32),
            pltpu.VMEM((2, _CHUNK, hidden), jnp.float32),
            pltpu.VMEM((2, _CHUNK, hidden), jnp.float32),
            pltpu.VMEM((2, _CHUNK, hidden), jnp.float32),
            pltpu.SemaphoreType.DMA,
            pltpu.SemaphoreType.DMA,
            pltpu.SemaphoreType.DMA,
            pltpu.SemaphoreType.DMA,
            pltpu.SemaphoreType.DMA,
            pltpu.SemaphoreType.DMA,
        ],
    )
    def k(widx_hbm, pidx_hbm, word_hbm, pos_hbm, out_hbm,
          widx_v, pidx_v, wbuf, pbuf, obuf,
          sem_w0, sem_w1, sem_p0, sem_p1, sem_o0, sem_o1):
        sem_w = (sem_w0, sem_w1)
        sem_p = (sem_p0, sem_p1)
        sem_o = (sem_o0, sem_o1)
        wid = lax.axis_index("s") * _NC + lax.axis_index("c")
        base = wid * rows_per_w

        pltpu.sync_copy(widx_hbm.at[pl.ds(base, rows_per_w)], widx_v)
        pltpu.sync_copy(pidx_hbm.at[pl.ds(base, rows_per_w)], pidx_v)

        def start_gathers(c, b):
            idx = pl.ds(c * _CHUNK, _CHUNK)
            pltpu.async_copy(word_hbm.at[widx_v.at[idx]], wbuf.at[b], sem_w[b])
            pltpu.async_copy(pos_hbm.at[pidx_v.at[idx]], pbuf.at[b], sem_p[b])

        def wait_gathers(c, b):
            idx = pl.ds(c * _CHUNK, _CHUNK)
            pltpu.make_async_copy(word_hbm.at[widx_v.at[idx]], wbuf.at[b], sem_w[b]).wait()
            pltpu.make_async_copy(pos_hbm.at[pidx_v.at[idx]], pbuf.at[b], sem_p[b]).wait()

        def out_slice(c):
            return out_hbm.at[pl.ds(base + c * _CHUNK, _CHUNK)]

        for b in range(2):
            start_gathers(b, b)

        def chunk_pair(g, _):
            for b in range(2):
                c = g * 2 + b
                wait_gathers(c, b)

                @pl.when(c >= 2)
                def _():
                    pltpu.make_async_copy(obuf.at[b], out_slice(c - 2), sem_o[b]).wait()

                def add_body(i, _):
                    r = i // vecs_per_row
                    j = (i % vecs_per_row) * 16
                    obuf[b, r, pl.ds(j, 16)] = (
                        wbuf[b, r, pl.ds(j, 16)] + pbuf[b, r, pl.ds(j, 16)]
                    )
                    return 0

                lax.fori_loop(0, _CHUNK * vecs_per_row, add_body, 0, unroll=8)

                pltpu.async_copy(obuf.at[b], out_slice(c), sem_o[b])

                @pl.when(c + 2 < n_chunks)
                def _():
                    start_gathers(c + 2, b)
            return 0

        lax.fori_loop(0, n_chunks // 2, chunk_pair, 0)

        for b in range(2):
            c = n_chunks - 2 + b
            pltpu.make_async_copy(obuf.at[b], out_slice(c), sem_o[b]).wait()

    return k


def kernel(input_ids, position_ids, word_embeddings, position_embeddings):
    batch, seq = input_ids.shape
    hidden = word_embeddings.shape[1]
    n_rows = batch * seq

    widx = jnp.transpose(input_ids, (1, 0)).reshape(n_rows).astype(jnp.int32)
    pidx = jnp.transpose(position_ids, (1, 0)).reshape(n_rows).astype(jnp.int32)

    rows_per_phase = n_rows // _PHASES
    s_per_phase = seq // _PHASES
    k = _make_sc_kernel(rows_per_phase, hidden)

    out = jnp.zeros((seq, batch, hidden), jnp.float32)
    for p in range(_PHASES):
        sl = slice(p * rows_per_phase, (p + 1) * rows_per_phase)
        flat = k(widx[sl], pidx[sl], word_embeddings, position_embeddings)
        out = lax.dynamic_update_slice(
            out, flat.reshape(s_per_phase, batch, hidden), (p * s_per_phase, 0, 0)
        )
    return out


# speedup vs baseline: 1.3792x; 1.2815x over previous
import functools

import jax
import jax.numpy as jnp
from jax import lax
from jax.experimental import pallas as pl
from jax.experimental.pallas import tpu as pltpu
from jax.experimental.pallas import tpu_sc as plsc

_INFO = plsc.get_sparse_core_info()
_NC = _INFO.num_cores
_NS = _INFO.num_subcores
_NW = _NC * _NS

_CHUNK = 16


def _make_sc_kernel(n_rows, hidden):
    rows_per_w = n_rows // _NW
    n_chunks = rows_per_w // _CHUNK
    vecs_per_row = hidden // 16
    mesh = plsc.VectorSubcoreMesh(core_axis_name="c", subcore_axis_name="s")

    @functools.partial(
        pl.kernel,
        mesh=mesh,
        out_type=jax.ShapeDtypeStruct((n_rows, hidden), jnp.float32),
        scratch_types=[
            pltpu.VMEM((rows_per_w,), jnp.int32),
            pltpu.VMEM((rows_per_w,), jnp.int32),
            pltpu.VMEM((2, _CHUNK, hidden), jnp.float32),
            pltpu.VMEM((2, _CHUNK, hidden), jnp.float32),
            pltpu.VMEM((2, _CHUNK, hidden), jnp.float32),
            pltpu.SemaphoreType.DMA,
            pltpu.SemaphoreType.DMA,
            pltpu.SemaphoreType.DMA,
            pltpu.SemaphoreType.DMA,
            pltpu.SemaphoreType.DMA,
            pltpu.SemaphoreType.DMA,
        ],
    )
    def k(widx_hbm, pidx_hbm, word_hbm, pos_hbm, out_hbm,
          widx_v, pidx_v, wbuf, pbuf, obuf,
          sem_w0, sem_w1, sem_p0, sem_p1, sem_o0, sem_o1):
        sem_w = (sem_w0, sem_w1)
        sem_p = (sem_p0, sem_p1)
        sem_o = (sem_o0, sem_o1)
        wid = lax.axis_index("s") * _NC + lax.axis_index("c")
        base = wid * rows_per_w

        pltpu.sync_copy(widx_hbm.at[pl.ds(base, rows_per_w)], widx_v)
        pltpu.sync_copy(pidx_hbm.at[pl.ds(base, rows_per_w)], pidx_v)

        def start_gathers(c, b):
            idx = pl.ds(c * _CHUNK, _CHUNK)
            pltpu.async_copy(word_hbm.at[widx_v.at[idx]], wbuf.at[b], sem_w[b])
            pltpu.async_copy(pos_hbm.at[pidx_v.at[idx]], pbuf.at[b], sem_p[b])

        def wait_gathers(c, b):
            idx = pl.ds(c * _CHUNK, _CHUNK)
            pltpu.make_async_copy(word_hbm.at[widx_v.at[idx]], wbuf.at[b], sem_w[b]).wait()
            pltpu.make_async_copy(pos_hbm.at[pidx_v.at[idx]], pbuf.at[b], sem_p[b]).wait()

        def out_slice(c):
            return out_hbm.at[pl.ds(base + c * _CHUNK, _CHUNK)]

        for b in range(2):
            start_gathers(b, b)

        def chunk_pair(g, _):
            for b in range(2):
                c = g * 2 + b
                wait_gathers(c, b)

                @pl.when(c >= 2)
                def _():
                    pltpu.make_async_copy(obuf.at[b], out_slice(c - 2), sem_o[b]).wait()

                def add_body(i, _):
                    r = i // vecs_per_row
                    j = (i % vecs_per_row) * 16
                    obuf[b, r, pl.ds(j, 16)] = (
                        wbuf[b, r, pl.ds(j, 16)] + pbuf[b, r, pl.ds(j, 16)]
                    )
                    return 0

                lax.fori_loop(0, _CHUNK * vecs_per_row, add_body, 0, unroll=8)

                pltpu.async_copy(obuf.at[b], out_slice(c), sem_o[b])

                @pl.when(c + 2 < n_chunks)
                def _():
                    start_gathers(c + 2, b)
            return 0

        lax.fori_loop(0, n_chunks // 2, chunk_pair, 0)

        for b in range(2):
            c = n_chunks - 2 + b
            pltpu.make_async_copy(obuf.at[b], out_slice(c), sem_o[b]).wait()

    return k


def kernel(input_ids, position_ids, word_embeddings, position_embeddings):
    batch, seq = input_ids.shape
    hidden = word_embeddings.shape[1]
    n_rows = batch * seq

    widx = jnp.transpose(input_ids, (1, 0)).reshape(n_rows).astype(jnp.int32)
    pidx = jnp.transpose(position_ids, (1, 0)).reshape(n_rows).astype(jnp.int32)

    k = _make_sc_kernel(n_rows, hidden)
    flat = k(widx, pidx, word_embeddings, position_embeddings)
    return flat.reshape(seq, batch, hidden)
